# Initial kernel scaffold; baseline (speedup 1.0000x reference)
import functools

import jax
import jax.numpy as jnp
from jax import lax
from jax.experimental import pallas as pl
from jax.experimental.pallas import tpu as pltpu
from jax.experimental.pallas import tpu_sc as plsc

D = 4
H = 128
OUT = 16
PACK = 32
LN_EPS = 1e-5

NC = 2
NS = 16
NW = NC * NS
CHUNK = 80


def _pack_body(x_ref, m1_ref, m2_ref, o_ref):
    xm = (x_ref[:, 0, :] + x_ref[:, 1, :] + x_ref[:, 2, :] + x_ref[:, 3, :]) * 0.25
    o_ref[...] = (
        jnp.dot(xm, m1_ref[...], preferred_element_type=jnp.float32)
        + jnp.dot(xm * xm, m2_ref[...], preferred_element_type=jnp.float32)
    )


def _pack_table(x4, m1, m2, blk):
    n = x4.shape[0]
    return pl.pallas_call(
        _pack_body,
        grid=(n // blk,),
        in_specs=[
            pl.BlockSpec((blk, D, H), lambda i: (i, 0, 0)),
            pl.BlockSpec((H, PACK), lambda i: (0, 0)),
            pl.BlockSpec((H, PACK), lambda i: (0, 0)),
        ],
        out_specs=pl.BlockSpec((blk, PACK), lambda i: (i, 0)),
        out_shape=jax.ShapeDtypeStruct((n, PACK), jnp.float32),
    )(x4, m1, m2)


def _sc_sheaf(tabx, tabe, row, col, consts, n_inc):
    per_w = n_inc // NW
    n_chunk = per_w // CHUNK
    mesh = plsc.VectorSubcoreMesh(core_axis_name="c", subcore_axis_name="s")

    @functools.partial(
        pl.kernel,
        mesh=mesh,
        out_type=jax.ShapeDtypeStruct((n_inc, OUT), jnp.float32),
        scratch_types=[
            pltpu.VMEM((per_w,), jnp.int32),
            pltpu.VMEM((per_w,), jnp.int32),
            pltpu.VMEM((CHUNK, 2, OUT), jnp.float32),
            pltpu.VMEM((CHUNK, 2, OUT), jnp.float32),
            pltpu.VMEM((CHUNK, OUT), jnp.float32),
            pltpu.VMEM((2, OUT), jnp.float32),
            pltpu.SemaphoreType.DMA,
            pltpu.SemaphoreType.DMA,
        ],
    )
    def k(tabx_hbm, tabe_hbm, row_hbm, col_hbm, c_hbm, out_hbm,
          rows_v, cols_v, bx_v, be_v, ob_v, cc_v, sem1, sem2):
        wid = lax.axis_index("s") * NC + lax.axis_index("c")
        base = wid * per_w
        pltpu.sync_copy(c_hbm, cc_v)
        pltpu.sync_copy(row_hbm.at[pl.ds(base, per_w)], rows_v)
        pltpu.sync_copy(col_hbm.at[pl.ds(base, per_w)], cols_v)
        cw = cc_v[0, :]
        cb = cc_v[1, :]

        def chunk_body(ci, _):
            off = ci * CHUNK
            cpx = pltpu.async_copy(
                tabx_hbm.at[rows_v.at[pl.ds(off, CHUNK)]], bx_v, sem1)
            cpe = pltpu.async_copy(
                tabe_hbm.at[cols_v.at[pl.ds(off, CHUNK)]], be_v, sem2)
            cpx.wait()
            cpe.wait()

            def inc_body(j, _):
                dv = bx_v[j, 0, :] + be_v[j, 0, :]
                s = bx_v[j, 1, 0] + be_v[j, 1, 0]
                q = bx_v[j, 1, 1] + be_v[j, 1, 1]
                mu = s * (1.0 / (2 * H))
                v = q * (1.0 / (2 * H)) - mu * mu + LN_EPS
                iv = lax.bitcast_convert_type(v, jnp.int32)
                iv = 0x5F3759DF - lax.shift_right_arithmetic(iv, 1)
                y = lax.bitcast_convert_type(iv, jnp.float32)
                hv = 0.5 * v
                y = y * (1.5 - hv * y * y)
                y = y * (1.5 - hv * y * y)
                y = y * (1.5 - hv * y * y)
                t = dv * y - (mu * y) * cw + cb
                ob_v[j, :] = 1.0 / (1.0 + jnp.exp(-t))
                return 0

            lax.fori_loop(0, CHUNK, inc_body, 0, unroll=2)
            pltpu.sync_copy(ob_v, out_hbm.at[pl.ds(base + off, CHUNK)])
            return 0

        lax.fori_loop(0, n_chunk, chunk_body, 0)

    return k(tabx, tabe, row, col, consts)


def kernel(x, e, hyperedge_index, node_types, hyperedge_types,
           ln_scale, ln_bias, W, b):
    n_nodes = x.shape[0] // D
    n_edges = e.shape[0] // D
    n_inc = hyperedge_index.shape[1]

    wp = ln_scale[:, None] * W
    zeros = jnp.zeros((H, OUT), jnp.float32)
    ones = jnp.ones((H, 1), jnp.float32)
    zcol = jnp.zeros((H, 1), jnp.float32)
    pad = jnp.zeros((H, PACK - OUT - 2), jnp.float32)
    m1x = jnp.concatenate([wp[:H], ones, zcol, pad], axis=1)
    m1e = jnp.concatenate([wp[H:], ones, zcol, pad], axis=1)
    m2 = jnp.concatenate([zeros, zcol, ones, pad], axis=1)

    tabx = _pack_table(x.reshape(n_nodes, D, H), m1x, m2, 1000)
    tabe = _pack_table(e.reshape(n_edges, D, H), m1e, m2, 1000)

    cw = ln_scale @ W
    cb = ln_bias @ W + b
    consts = jnp.stack([cw, cb], axis=0)

    row = hyperedge_index[0].astype(jnp.int32)
    col = hyperedge_index[1].astype(jnp.int32)

    tabx3 = tabx.reshape(n_nodes, 2, OUT)
    tabe3 = tabe.reshape(n_edges, 2, OUT)
    return _sc_sheaf(tabx3, tabe3, row, col, consts, n_inc)


# speedup vs baseline: 2.4988x; 2.4988x over previous
import functools

import jax
import jax.numpy as jnp
from jax import lax
from jax.experimental import pallas as pl
from jax.experimental.pallas import tpu as pltpu
from jax.experimental.pallas import tpu_sc as plsc

D = 4
H = 128
OUT = 16
PACK = 32
LN_EPS = 1e-5

NC = 2
NS = 16
NW = NC * NS
CHUNK = 80


def _pack_body(x_ref, m1_ref, m2_ref, o_ref):
    xm = (x_ref[:, 0, :] + x_ref[:, 1, :] + x_ref[:, 2, :] + x_ref[:, 3, :]) * 0.25
    o_ref[...] = (
        jnp.dot(xm, m1_ref[...], preferred_element_type=jnp.float32)
        + jnp.dot(xm * xm, m2_ref[...], preferred_element_type=jnp.float32)
    )


def _pack_table(x4, m1, m2, blk):
    n = x4.shape[0]
    return pl.pallas_call(
        _pack_body,
        grid=(n // blk,),
        in_specs=[
            pl.BlockSpec((blk, D, H), lambda i: (i, 0, 0)),
            pl.BlockSpec((H, PACK), lambda i: (0, 0)),
            pl.BlockSpec((H, PACK), lambda i: (0, 0)),
        ],
        out_specs=pl.BlockSpec((blk, PACK), lambda i: (i, 0)),
        out_shape=jax.ShapeDtypeStruct((n, PACK), jnp.float32),
    )(x4, m1, m2)


def _sc_sheaf(tabx, tabe, row, col, consts, n_inc):
    per_w = n_inc // NW
    n_chunk = per_w // CHUNK
    mesh = plsc.VectorSubcoreMesh(core_axis_name="c", subcore_axis_name="s")

    @functools.partial(
        pl.kernel,
        mesh=mesh,
        out_type=jax.ShapeDtypeStruct((n_inc, OUT), jnp.float32),
        compiler_params=pltpu.CompilerParams(use_tc_tiling_on_sc=False),
        scratch_types=[
            pltpu.VMEM((per_w,), jnp.int32),
            pltpu.VMEM((per_w,), jnp.int32),
            pltpu.VMEM((CHUNK, PACK), jnp.float32),
            pltpu.VMEM((CHUNK, PACK), jnp.float32),
            pltpu.VMEM((CHUNK, OUT), jnp.float32),
            pltpu.VMEM((2, OUT), jnp.float32),
            pltpu.SemaphoreType.DMA,
            pltpu.SemaphoreType.DMA,
        ],
    )
    def k(tabx_hbm, tabe_hbm, row_hbm, col_hbm, c_hbm, out_hbm,
          rows_v, cols_v, bx_v, be_v, ob_v, cc_v, sem1, sem2):
        wid = lax.axis_index("s") * NC + lax.axis_index("c")
        base = wid * per_w
        pltpu.sync_copy(c_hbm, cc_v)
        pltpu.sync_copy(row_hbm.at[pl.ds(base, per_w)], rows_v)
        pltpu.sync_copy(col_hbm.at[pl.ds(base, per_w)], cols_v)
        cw = cc_v[0, :]
        cb = cc_v[1, :]

        def chunk_body(ci, _):
            off = ci * CHUNK
            cpx = pltpu.async_copy(
                tabx_hbm.at[rows_v.at[pl.ds(off, CHUNK)]], bx_v, sem1)
            cpe = pltpu.async_copy(
                tabe_hbm.at[cols_v.at[pl.ds(off, CHUNK)]], be_v, sem2)
            cpx.wait()
            cpe.wait()

            def inc_body(j, _):
                dv = bx_v[j, 0:OUT] + be_v[j, 0:OUT]
                st = bx_v[j, OUT:PACK] + be_v[j, OUT:PACK]
                s = st[0]
                q = st[1]
                mu = s * (1.0 / (2 * H))
                v = q * (1.0 / (2 * H)) - mu * mu + LN_EPS
                iv = lax.bitcast_convert_type(v, jnp.int32)
                iv = 0x5F3759DF - lax.shift_right_arithmetic(iv, 1)
                y = lax.bitcast_convert_type(iv, jnp.float32)
                hv = 0.5 * v
                y = y * (1.5 - hv * y * y)
                y = y * (1.5 - hv * y * y)
                y = y * (1.5 - hv * y * y)
                t = dv * y - (mu * y) * cw + cb
                ob_v[j, :] = 1.0 / (1.0 + jnp.exp(-t))
                return 0

            lax.fori_loop(0, CHUNK, inc_body, 0, unroll=2)
            pltpu.sync_copy(ob_v, out_hbm.at[pl.ds(base + off, CHUNK)])
            return 0

        lax.fori_loop(0, n_chunk, chunk_body, 0)

    return k(tabx, tabe, row, col, consts)


def kernel(x, e, hyperedge_index, node_types, hyperedge_types,
           ln_scale, ln_bias, W, b):
    n_nodes = x.shape[0] // D
    n_edges = e.shape[0] // D
    n_inc = hyperedge_index.shape[1]

    wp = ln_scale[:, None] * W
    zeros = jnp.zeros((H, OUT), jnp.float32)
    ones = jnp.ones((H, 1), jnp.float32)
    zcol = jnp.zeros((H, 1), jnp.float32)
    pad = jnp.zeros((H, PACK - OUT - 2), jnp.float32)
    m1x = jnp.concatenate([wp[:H], ones, zcol, pad], axis=1)
    m1e = jnp.concatenate([wp[H:], ones, zcol, pad], axis=1)
    m2 = jnp.concatenate([zeros, zcol, ones, pad], axis=1)

    tabx = _pack_table(x.reshape(n_nodes, D, H), m1x, m2, 1000)
    tabe = _pack_table(e.reshape(n_edges, D, H), m1e, m2, 1000)

    cw = ln_scale @ W
    cb = ln_bias @ W + b
    consts = jnp.stack([cw, cb], axis=0)

    row = hyperedge_index[0].astype(jnp.int32)
    col = hyperedge_index[1].astype(jnp.int32)

    return _sc_sheaf(tabx, tabe, row, col, consts, n_inc)


# speedup vs baseline: 2.7926x; 1.1176x over previous
import functools

import jax
import jax.numpy as jnp
from jax import lax
from jax.experimental import pallas as pl
from jax.experimental.pallas import tpu as pltpu
from jax.experimental.pallas import tpu_sc as plsc

D = 4
H = 128
OUT = 16
PACK = 32
LN_EPS = 1e-5

NC = 2
NS = 16
NW = NC * NS
CHUNK = 80


def _pack_body(x_ref, m1_ref, m2_ref, o_ref):
    xm = (x_ref[:, 0, :] + x_ref[:, 1, :] + x_ref[:, 2, :] + x_ref[:, 3, :]) * 0.25
    o_ref[...] = (
        jnp.dot(xm, m1_ref[...], preferred_element_type=jnp.float32)
        + jnp.dot(xm * xm, m2_ref[...], preferred_element_type=jnp.float32)
    )


def _pack_table(x4, m1, m2, blk):
    n = x4.shape[0]
    return pl.pallas_call(
        _pack_body,
        grid=(n // blk,),
        in_specs=[
            pl.BlockSpec((blk, D, H), lambda i: (i, 0, 0)),
            pl.BlockSpec((H, PACK), lambda i: (0, 0)),
            pl.BlockSpec((H, PACK), lambda i: (0, 0)),
        ],
        out_specs=pl.BlockSpec((blk, PACK), lambda i: (i, 0)),
        out_shape=jax.ShapeDtypeStruct((n, PACK), jnp.float32),
    )(x4, m1, m2)


def _sc_sheaf(tabx, tabe, row, col, consts, n_inc):
    per_w = n_inc // NW
    n_chunk = per_w // CHUNK
    mesh = plsc.VectorSubcoreMesh(core_axis_name="c", subcore_axis_name="s")

    @functools.partial(
        pl.kernel,
        mesh=mesh,
        out_type=jax.ShapeDtypeStruct((n_inc, OUT), jnp.float32),
        compiler_params=pltpu.CompilerParams(use_tc_tiling_on_sc=False),
        scratch_types=[
            pltpu.VMEM((per_w,), jnp.int32),
            pltpu.VMEM((per_w,), jnp.int32),
            pltpu.VMEM((CHUNK, PACK), jnp.float32),
            pltpu.VMEM((CHUNK, PACK), jnp.float32),
            pltpu.VMEM((CHUNK, PACK), jnp.float32),
            pltpu.VMEM((CHUNK, PACK), jnp.float32),
            pltpu.VMEM((CHUNK, OUT), jnp.float32),
            pltpu.VMEM((CHUNK, OUT), jnp.float32),
            pltpu.VMEM((2, OUT), jnp.float32),
            pltpu.SemaphoreType.DMA,
            pltpu.SemaphoreType.DMA,
        ],
    )
    def k(tabx_hbm, tabe_hbm, row_hbm, col_hbm, c_hbm, out_hbm,
          rows_v, cols_v, bxa_v, bea_v, bxb_v, beb_v, oba_v, obb_v,
          cc_v, sema, semb):
        wid = lax.axis_index("s") * NC + lax.axis_index("c")
        base = wid * per_w
        pltpu.sync_copy(c_hbm, cc_v)
        pltpu.sync_copy(row_hbm.at[pl.ds(base, per_w)], rows_v)
        pltpu.sync_copy(col_hbm.at[pl.ds(base, per_w)], cols_v)
        cw = cc_v[0, :]
        cb = cc_v[1, :]

        def fire(ci, bx_v, be_v, sem):
            off = ci * CHUNK
            cpx = pltpu.async_copy(
                tabx_hbm.at[rows_v.at[pl.ds(off, CHUNK)]], bx_v, sem)
            cpe = pltpu.async_copy(
                tabe_hbm.at[cols_v.at[pl.ds(off, CHUNK)]], be_v, sem)
            return cpx, cpe

        def compute(ci, bx_v, be_v, ob_v):
            def inc_body(j, _):
                dv = bx_v[j, 0:OUT] + be_v[j, 0:OUT]
                st = bx_v[j, OUT:PACK] + be_v[j, OUT:PACK]
                s = st[0]
                q = st[1]
                mu = s * (1.0 / (2 * H))
                v = q * (1.0 / (2 * H)) - mu * mu + LN_EPS
                iv = lax.bitcast_convert_type(v, jnp.int32)
                iv = 0x5F3759DF - lax.shift_right_arithmetic(iv, 1)
                y = lax.bitcast_convert_type(iv, jnp.float32)
                hv = 0.5 * v
                y = y * (1.5 - hv * y * y)
                y = y * (1.5 - hv * y * y)
                y = y * (1.5 - hv * y * y)
                t = dv * y - (mu * y) * cw + cb
                ob_v[j, :] = 1.0 / (1.0 + jnp.exp(-t))
                return 0

            lax.fori_loop(0, CHUNK, inc_body, 0, unroll=2)
            pltpu.sync_copy(ob_v, out_hbm.at[pl.ds(base + ci * CHUNK, CHUNK)])

        def drain(bx_v, be_v, sem):
            pltpu.make_async_copy(tabx_hbm.at[pl.ds(0, CHUNK)], bx_v, sem).wait()
            pltpu.make_async_copy(tabe_hbm.at[pl.ds(0, CHUNK)], be_v, sem).wait()

        fire(0, bxa_v, bea_v, sema)

        def pair_body(p, _):
            fire(2 * p + 1, bxb_v, beb_v, semb)
            drain(bxa_v, bea_v, sema)
            compute(2 * p, bxa_v, bea_v, oba_v)
            fire(2 * p + 2, bxa_v, bea_v, sema)
            drain(bxb_v, beb_v, semb)
            compute(2 * p + 1, bxb_v, beb_v, obb_v)
            return 0

        lax.fori_loop(0, (n_chunk - 1) // 2, pair_body, 0)
        drain(bxa_v, bea_v, sema)
        compute(n_chunk - 1, bxa_v, bea_v, oba_v)

    return k(tabx, tabe, row, col, consts)


def kernel(x, e, hyperedge_index, node_types, hyperedge_types,
           ln_scale, ln_bias, W, b):
    n_nodes = x.shape[0] // D
    n_edges = e.shape[0] // D
    n_inc = hyperedge_index.shape[1]

    wp = ln_scale[:, None] * W
    zeros = jnp.zeros((H, OUT), jnp.float32)
    ones = jnp.ones((H, 1), jnp.float32)
    zcol = jnp.zeros((H, 1), jnp.float32)
    pad = jnp.zeros((H, PACK - OUT - 2), jnp.float32)
    m1x = jnp.concatenate([wp[:H], ones, zcol, pad], axis=1)
    m1e = jnp.concatenate([wp[H:], ones, zcol, pad], axis=1)
    m2 = jnp.concatenate([zeros, zcol, ones, pad], axis=1)

    tabx = _pack_table(x.reshape(n_nodes, D, H), m1x, m2, 1000)
    tabe = _pack_table(e.reshape(n_edges, D, H), m1e, m2, 1000)

    cw = ln_scale @ W
    cb = ln_bias @ W + b
    consts = jnp.stack([cw, cb], axis=0)

    row = hyperedge_index[0].astype(jnp.int32)
    col = hyperedge_index[1].astype(jnp.int32)

    return _sc_sheaf(tabx, tabe, row, col, consts, n_inc)


# speedup vs baseline: 3.1351x; 1.1227x over previous
import functools

import jax
import jax.numpy as jnp
from jax import lax
from jax.experimental import pallas as pl
from jax.experimental.pallas import tpu as pltpu
from jax.experimental.pallas import tpu_sc as plsc

D = 4
H = 128
OUT = 16
PACK = 32
LN_EPS = 1e-5

NC = 2
NS = 16
NW = NC * NS
CHUNK = 80


def _pack_body(x_ref, m1_ref, m2_ref, o_ref):
    xm = (x_ref[:, 0, :] + x_ref[:, 1, :] + x_ref[:, 2, :] + x_ref[:, 3, :]) * 0.25
    o_ref[...] = (
        jnp.dot(xm, m1_ref[...], preferred_element_type=jnp.float32)
        + jnp.dot(xm * xm, m2_ref[...], preferred_element_type=jnp.float32)
    )


def _pack_table(x4, m1, m2, blk):
    n = x4.shape[0]
    return pl.pallas_call(
        _pack_body,
        grid=(n // blk,),
        in_specs=[
            pl.BlockSpec((blk, D, H), lambda i: (i, 0, 0)),
            pl.BlockSpec((H, PACK), lambda i: (0, 0)),
            pl.BlockSpec((H, PACK), lambda i: (0, 0)),
        ],
        out_specs=pl.BlockSpec((blk, PACK), lambda i: (i, 0)),
        out_shape=jax.ShapeDtypeStruct((n, PACK), jnp.float32),
    )(x4, m1, m2)


def _sc_sheaf(tabx, tabe, row, col, consts, n_inc):
    per_w = n_inc // NW
    n_chunk = per_w // CHUNK
    mesh = plsc.VectorSubcoreMesh(core_axis_name="c", subcore_axis_name="s")

    @functools.partial(
        pl.kernel,
        mesh=mesh,
        out_type=jax.ShapeDtypeStruct((n_inc, OUT), jnp.float32),
        compiler_params=pltpu.CompilerParams(
            use_tc_tiling_on_sc=False, needs_layout_passes=False),
        scratch_types=[
            pltpu.VMEM((per_w,), jnp.int32),
            pltpu.VMEM((per_w,), jnp.int32),
            pltpu.VMEM((CHUNK, PACK), jnp.float32),
            pltpu.VMEM((CHUNK, PACK), jnp.float32),
            pltpu.VMEM((CHUNK, PACK), jnp.float32),
            pltpu.VMEM((CHUNK, PACK), jnp.float32),
            pltpu.VMEM((CHUNK, OUT), jnp.float32),
            pltpu.VMEM((CHUNK, OUT), jnp.float32),
            pltpu.VMEM((2, OUT), jnp.float32),
            pltpu.SemaphoreType.DMA,
            pltpu.SemaphoreType.DMA,
        ],
    )
    def k(tabx_hbm, tabe_hbm, row_hbm, col_hbm, c_hbm, out_hbm,
          rows_v, cols_v, bxa_v, bea_v, bxb_v, beb_v, oba_v, obb_v,
          cc_v, sema, semb):
        wid = lax.axis_index("s") * NC + lax.axis_index("c")
        base = wid * per_w
        pltpu.sync_copy(c_hbm, cc_v)
        pltpu.sync_copy(row_hbm.at[pl.ds(base, per_w)], rows_v)
        pltpu.sync_copy(col_hbm.at[pl.ds(base, per_w)], cols_v)
        cw = cc_v[0, :]
        cb = cc_v[1, :]

        def fire(ci, bx_v, be_v, sem):
            off = ci * CHUNK
            cpx = pltpu.async_copy(
                tabx_hbm.at[rows_v.at[pl.ds(off, CHUNK)]], bx_v, sem)
            cpe = pltpu.async_copy(
                tabe_hbm.at[cols_v.at[pl.ds(off, CHUNK)]], be_v, sem)
            return cpx, cpe

        lane0 = jnp.full((16,), 0, jnp.int32)
        lane1 = jnp.full((16,), 1, jnp.int32)

        def compute(ci, bx_v, be_v, ob_v):
            def inc_body(j, _):
                st = bx_v[j, OUT:PACK] + be_v[j, OUT:PACK]
                s = st.at[lane0].get(mode="promise_in_bounds")
                q = st.at[lane1].get(mode="promise_in_bounds")
                mu = s * (1.0 / (2 * H))
                v = q * (1.0 / (2 * H)) - mu * mu + LN_EPS
                iv = plsc.bitcast(v, jnp.int32)
                iv = 0x5F3759DF - lax.shift_right_arithmetic(iv, 1)
                y = plsc.bitcast(iv, jnp.float32)
                hv = 0.5 * v
                y = y * (1.5 - hv * y * y)
                y = y * (1.5 - hv * y * y)
                y = y * (1.5 - hv * y * y)
                dv = bx_v[j, 0:OUT] + be_v[j, 0:OUT]
                t = dv * y - (mu * y) * cw + cb
                ob_v[j, :] = 1.0 / (1.0 + jnp.exp(-t))
                return 0

            lax.fori_loop(0, CHUNK, inc_body, 0, unroll=4)
            pltpu.sync_copy(ob_v, out_hbm.at[pl.ds(base + ci * CHUNK, CHUNK)])

        def drain(bx_v, be_v, sem):
            pltpu.make_async_copy(tabx_hbm.at[pl.ds(0, CHUNK)], bx_v, sem).wait()
            pltpu.make_async_copy(tabe_hbm.at[pl.ds(0, CHUNK)], be_v, sem).wait()

        fire(0, bxa_v, bea_v, sema)

        def pair_body(p, _):
            fire(2 * p + 1, bxb_v, beb_v, semb)
            drain(bxa_v, bea_v, sema)
            compute(2 * p, bxa_v, bea_v, oba_v)
            fire(2 * p + 2, bxa_v, bea_v, sema)
            drain(bxb_v, beb_v, semb)
            compute(2 * p + 1, bxb_v, beb_v, obb_v)
            return 0

        lax.fori_loop(0, (n_chunk - 1) // 2, pair_body, 0)
        drain(bxa_v, bea_v, sema)
        compute(n_chunk - 1, bxa_v, bea_v, oba_v)

    return k(tabx, tabe, row, col, consts)


def kernel(x, e, hyperedge_index, node_types, hyperedge_types,
           ln_scale, ln_bias, W, b):
    n_nodes = x.shape[0] // D
    n_edges = e.shape[0] // D
    n_inc = hyperedge_index.shape[1]

    wp = ln_scale[:, None] * W
    zeros = jnp.zeros((H, OUT), jnp.float32)
    ones = jnp.ones((H, 1), jnp.float32)
    zcol = jnp.zeros((H, 1), jnp.float32)
    pad = jnp.zeros((H, PACK - OUT - 2), jnp.float32)
    m1x = jnp.concatenate([wp[:H], ones, zcol, pad], axis=1)
    m1e = jnp.concatenate([wp[H:], ones, zcol, pad], axis=1)
    m2 = jnp.concatenate([zeros, zcol, ones, pad], axis=1)

    tabx = _pack_table(x.reshape(n_nodes, D, H), m1x, m2, 1000)
    tabe = _pack_table(e.reshape(n_edges, D, H), m1e, m2, 1000)

    cw = ln_scale @ W
    cb = ln_bias @ W + b
    consts = jnp.stack([cw, cb], axis=0)

    row = hyperedge_index[0].astype(jnp.int32)
    col = hyperedge_index[1].astype(jnp.int32)

    return _sc_sheaf(tabx, tabe, row, col, consts, n_inc)


# speedup vs baseline: 6.7395x; 2.1497x over previous
import functools

import jax
import jax.numpy as jnp
from jax import lax
from jax.experimental import pallas as pl
from jax.experimental.pallas import tpu as pltpu
from jax.experimental.pallas import tpu_sc as plsc

D = 4
H = 128
OUT = 16
PACK = 32
LN_EPS = 1e-5

NC = 2
NS = 16
NW = NC * NS
CHUNK = 80


def _pack_body(x_ref, m1_ref, m2_ref, o_ref):
    xm = (x_ref[:, 0, :] + x_ref[:, 1, :] + x_ref[:, 2, :] + x_ref[:, 3, :]) * 0.25
    o_ref[...] = (
        jnp.dot(xm, m1_ref[...], preferred_element_type=jnp.float32)
        + jnp.dot(xm * xm, m2_ref[...], preferred_element_type=jnp.float32)
    )


def _pack_table(x4, m1, m2, blk):
    n = x4.shape[0]
    return pl.pallas_call(
        _pack_body,
        grid=(n // blk,),
        in_specs=[
            pl.BlockSpec((blk, D, H), lambda i: (i, 0, 0)),
            pl.BlockSpec((H, PACK), lambda i: (0, 0)),
            pl.BlockSpec((H, PACK), lambda i: (0, 0)),
        ],
        out_specs=pl.BlockSpec((blk, PACK), lambda i: (i, 0)),
        out_shape=jax.ShapeDtypeStruct((n, PACK), jnp.float32),
    )(x4, m1, m2)


def _sc_sheaf(tabx, tabe, row, col, consts, n_inc):
    per_w = n_inc // NW
    n_chunk = per_w // CHUNK
    mesh = plsc.VectorSubcoreMesh(core_axis_name="c", subcore_axis_name="s")

    @functools.partial(
        pl.kernel,
        mesh=mesh,
        out_type=jax.ShapeDtypeStruct((n_inc, OUT), jnp.float32),
        compiler_params=pltpu.CompilerParams(
            use_tc_tiling_on_sc=False, needs_layout_passes=False),
        scratch_types=[
            pltpu.VMEM((per_w,), jnp.int32),
            pltpu.VMEM((per_w,), jnp.int32),
            pltpu.VMEM((CHUNK, PACK), jnp.float32),
            pltpu.VMEM((CHUNK, PACK), jnp.float32),
            pltpu.VMEM((CHUNK, PACK), jnp.float32),
            pltpu.VMEM((CHUNK, PACK), jnp.float32),
            pltpu.VMEM((CHUNK, OUT), jnp.float32),
            pltpu.VMEM((CHUNK, OUT), jnp.float32),
            pltpu.VMEM((2, OUT), jnp.float32),
            pltpu.SemaphoreType.DMA,
            pltpu.SemaphoreType.DMA,
        ],
    )
    def k(tabx_hbm, tabe_hbm, row_hbm, col_hbm, c_hbm, out_hbm,
          rows_v, cols_v, bxa_v, bea_v, bxb_v, beb_v, oba_v, obb_v,
          cc_v, sema, semb):
        wid = lax.axis_index("s") * NC + lax.axis_index("c")
        base = wid * per_w
        pltpu.sync_copy(c_hbm, cc_v)
        pltpu.sync_copy(row_hbm.at[pl.ds(base, per_w)], rows_v)
        pltpu.sync_copy(col_hbm.at[pl.ds(base, per_w)], cols_v)
        cw = cc_v[0, :]
        cb = cc_v[1, :]

        def fire(ci, bx_v, be_v, sem):
            off = ci * CHUNK
            cpx = pltpu.async_copy(
                tabx_hbm.at[rows_v.at[pl.ds(off, CHUNK)]], bx_v, sem)
            cpe = pltpu.async_copy(
                tabe_hbm.at[cols_v.at[pl.ds(off, CHUNK)]], be_v, sem)
            return cpx, cpe

        lane0 = jnp.full((16,), 0, jnp.int32)
        lane1 = jnp.full((16,), 1, jnp.int32)

        def compute(ci, bx_v, be_v, ob_v):
            @plsc.parallel_loop(0, CHUNK, unroll=4)
            def inc_body(j):
                st = bx_v[j, OUT:PACK] + be_v[j, OUT:PACK]
                s = st.at[lane0].get(mode="promise_in_bounds")
                q = st.at[lane1].get(mode="promise_in_bounds")
                mu = s * (1.0 / (2 * H))
                v = q * (1.0 / (2 * H)) - mu * mu + LN_EPS
                iv = plsc.bitcast(v, jnp.int32)
                iv = 0x5F3759DF - lax.shift_right_arithmetic(iv, 1)
                y = plsc.bitcast(iv, jnp.float32)
                hv = 0.5 * v
                y = y * (1.5 - hv * y * y)
                y = y * (1.5 - hv * y * y)
                y = y * (1.5 - hv * y * y)
                dv = bx_v[j, 0:OUT] + be_v[j, 0:OUT]
                t = dv * y - (mu * y) * cw + cb
                ob_v[j, :] = 1.0 / (1.0 + jnp.exp(-t))

            pltpu.sync_copy(ob_v, out_hbm.at[pl.ds(base + ci * CHUNK, CHUNK)])

        def drain(bx_v, be_v, sem):
            pltpu.make_async_copy(tabx_hbm.at[pl.ds(0, CHUNK)], bx_v, sem).wait()
            pltpu.make_async_copy(tabe_hbm.at[pl.ds(0, CHUNK)], be_v, sem).wait()

        fire(0, bxa_v, bea_v, sema)

        def pair_body(p, _):
            fire(2 * p + 1, bxb_v, beb_v, semb)
            drain(bxa_v, bea_v, sema)
            compute(2 * p, bxa_v, bea_v, oba_v)
            fire(2 * p + 2, bxa_v, bea_v, sema)
            drain(bxb_v, beb_v, semb)
            compute(2 * p + 1, bxb_v, beb_v, obb_v)
            return 0

        lax.fori_loop(0, (n_chunk - 1) // 2, pair_body, 0)
        drain(bxa_v, bea_v, sema)
        compute(n_chunk - 1, bxa_v, bea_v, oba_v)

    return k(tabx, tabe, row, col, consts)


def kernel(x, e, hyperedge_index, node_types, hyperedge_types,
           ln_scale, ln_bias, W, b):
    n_nodes = x.shape[0] // D
    n_edges = e.shape[0] // D
    n_inc = hyperedge_index.shape[1]

    wp = ln_scale[:, None] * W
    zeros = jnp.zeros((H, OUT), jnp.float32)
    ones = jnp.ones((H, 1), jnp.float32)
    zcol = jnp.zeros((H, 1), jnp.float32)
    pad = jnp.zeros((H, PACK - OUT - 2), jnp.float32)
    m1x = jnp.concatenate([wp[:H], ones, zcol, pad], axis=1)
    m1e = jnp.concatenate([wp[H:], ones, zcol, pad], axis=1)
    m2 = jnp.concatenate([zeros, zcol, ones, pad], axis=1)

    tabx = _pack_table(x.reshape(n_nodes, D, H), m1x, m2, 1000)
    tabe = _pack_table(e.reshape(n_edges, D, H), m1e, m2, 1000)

    cw = ln_scale @ W
    cb = ln_bias @ W + b
    consts = jnp.stack([cw, cb], axis=0)

    row = hyperedge_index[0].astype(jnp.int32)
    col = hyperedge_index[1].astype(jnp.int32)

    return _sc_sheaf(tabx, tabe, row, col, consts, n_inc)


# speedup vs baseline: 6.8930x; 1.0228x over previous
import functools

import jax
import jax.numpy as jnp
from jax import lax
from jax.experimental import pallas as pl
from jax.experimental.pallas import tpu as pltpu
from jax.experimental.pallas import tpu_sc as plsc

D = 4
H = 128
OUT = 16
PACK = 32
LN_EPS = 1e-5

NC = 2
NS = 16
NW = NC * NS
CHUNK = 80


def _pack_body(x_ref, m1_ref, m2_ref, o_ref):
    xm = (x_ref[:, 0, :] + x_ref[:, 1, :] + x_ref[:, 2, :] + x_ref[:, 3, :]) * 0.25
    o_ref[...] = (
        jnp.dot(xm, m1_ref[...], preferred_element_type=jnp.float32)
        + jnp.dot(xm * xm, m2_ref[...], preferred_element_type=jnp.float32)
    )


def _pack_table(x4, m1, m2, blk):
    n = x4.shape[0]
    return pl.pallas_call(
        _pack_body,
        grid=(n // blk,),
        in_specs=[
            pl.BlockSpec((blk, D, H), lambda i: (i, 0, 0)),
            pl.BlockSpec((H, PACK), lambda i: (0, 0)),
            pl.BlockSpec((H, PACK), lambda i: (0, 0)),
        ],
        out_specs=pl.BlockSpec((blk, PACK), lambda i: (i, 0)),
        out_shape=jax.ShapeDtypeStruct((n, PACK), jnp.float32),
    )(x4, m1, m2)


def _sc_sheaf(tabx, tabe, row, col, consts, n_inc):
    per_w = n_inc // NW
    n_chunk = per_w // CHUNK
    mesh = plsc.VectorSubcoreMesh(core_axis_name="c", subcore_axis_name="s")

    @functools.partial(
        pl.kernel,
        mesh=mesh,
        out_type=jax.ShapeDtypeStruct((n_inc, OUT), jnp.float32),
        compiler_params=pltpu.CompilerParams(
            use_tc_tiling_on_sc=False, needs_layout_passes=False),
        scratch_types=[
            pltpu.VMEM((per_w,), jnp.int32),
            pltpu.VMEM((per_w,), jnp.int32),
            pltpu.VMEM((CHUNK, PACK), jnp.float32),
            pltpu.VMEM((CHUNK, PACK), jnp.float32),
            pltpu.VMEM((CHUNK, PACK), jnp.float32),
            pltpu.VMEM((CHUNK, PACK), jnp.float32),
            pltpu.VMEM((CHUNK, OUT), jnp.float32),
            pltpu.VMEM((CHUNK, OUT), jnp.float32),
            pltpu.VMEM((2, OUT), jnp.float32),
            pltpu.SemaphoreType.DMA,
            pltpu.SemaphoreType.DMA,
            pltpu.SemaphoreType.DMA,
            pltpu.SemaphoreType.DMA,
        ],
    )
    def k(tabx_hbm, tabe_hbm, row_hbm, col_hbm, c_hbm, out_hbm,
          rows_v, cols_v, bxa_v, bea_v, bxb_v, beb_v, oba_v, obb_v,
          cc_v, sema, semb, semoa, semob):
        wid = lax.axis_index("s") * NC + lax.axis_index("c")
        base = wid * per_w
        pltpu.sync_copy(c_hbm, cc_v)
        pltpu.sync_copy(row_hbm.at[pl.ds(base, per_w)], rows_v)
        pltpu.sync_copy(col_hbm.at[pl.ds(base, per_w)], cols_v)
        cw = cc_v[0, :]
        cb = cc_v[1, :]

        def fire(ci, bx_v, be_v, sem):
            off = ci * CHUNK
            cpx = pltpu.async_copy(
                tabx_hbm.at[rows_v.at[pl.ds(off, CHUNK)]], bx_v, sem)
            cpe = pltpu.async_copy(
                tabe_hbm.at[cols_v.at[pl.ds(off, CHUNK)]], be_v, sem)
            return cpx, cpe

        lane0 = jnp.full((16,), 0, jnp.int32)
        lane1 = jnp.full((16,), 1, jnp.int32)

        def drain_out(ob_v, semo):
            pltpu.make_async_copy(
                tabx_hbm.at[pl.ds(0, CHUNK), 0:OUT], ob_v, semo).wait()

        def compute(ci, bx_v, be_v, ob_v, semo):
            @pl.when(ci >= 2)
            def _():
                drain_out(ob_v, semo)

            @plsc.parallel_loop(0, CHUNK, unroll=8)
            def inc_body(j):
                st = bx_v[j, OUT:PACK] + be_v[j, OUT:PACK]
                s = st.at[lane0].get(mode="promise_in_bounds")
                q = st.at[lane1].get(mode="promise_in_bounds")
                mu = s * (1.0 / (2 * H))
                v = q * (1.0 / (2 * H)) - mu * mu + LN_EPS
                iv = plsc.bitcast(v, jnp.int32)
                iv = 0x5F3759DF - lax.shift_right_arithmetic(iv, 1)
                y = plsc.bitcast(iv, jnp.float32)
                hv = 0.5 * v
                y = y * (1.5 - hv * y * y)
                y = y * (1.5 - hv * y * y)
                dv = bx_v[j, 0:OUT] + be_v[j, 0:OUT]
                t = dv * y - (mu * y) * cw + cb
                ob_v[j, :] = 1.0 / (1.0 + jnp.exp(-t))

            pltpu.async_copy(
                ob_v, out_hbm.at[pl.ds(base + ci * CHUNK, CHUNK)], semo)

        def drain(bx_v, be_v, sem):
            pltpu.make_async_copy(tabx_hbm.at[pl.ds(0, CHUNK)], bx_v, sem).wait()
            pltpu.make_async_copy(tabe_hbm.at[pl.ds(0, CHUNK)], be_v, sem).wait()

        fire(0, bxa_v, bea_v, sema)

        def pair_body(p, _):
            fire(2 * p + 1, bxb_v, beb_v, semb)
            drain(bxa_v, bea_v, sema)
            compute(2 * p, bxa_v, bea_v, oba_v, semoa)
            fire(2 * p + 2, bxa_v, bea_v, sema)
            drain(bxb_v, beb_v, semb)
            compute(2 * p + 1, bxb_v, beb_v, obb_v, semob)
            return 0

        lax.fori_loop(0, (n_chunk - 1) // 2, pair_body, 0)
        drain(bxa_v, bea_v, sema)
        compute(n_chunk - 1, bxa_v, bea_v, oba_v, semoa)
        drain_out(oba_v, semoa)
        drain_out(obb_v, semob)

    return k(tabx, tabe, row, col, consts)


def kernel(x, e, hyperedge_index, node_types, hyperedge_types,
           ln_scale, ln_bias, W, b):
    n_nodes = x.shape[0] // D
    n_edges = e.shape[0] // D
    n_inc = hyperedge_index.shape[1]

    wp = ln_scale[:, None] * W
    zeros = jnp.zeros((H, OUT), jnp.float32)
    ones = jnp.ones((H, 1), jnp.float32)
    zcol = jnp.zeros((H, 1), jnp.float32)
    pad = jnp.zeros((H, PACK - OUT - 2), jnp.float32)
    m1x = jnp.concatenate([wp[:H], ones, zcol, pad], axis=1)
    m1e = jnp.concatenate([wp[H:], ones, zcol, pad], axis=1)
    m2 = jnp.concatenate([zeros, zcol, ones, pad], axis=1)

    tabx = _pack_table(x.reshape(n_nodes, D, H), m1x, m2, 1000)
    tabe = _pack_table(e.reshape(n_edges, D, H), m1e, m2, 1000)

    cw = ln_scale @ W
    cb = ln_bias @ W + b
    consts = jnp.stack([cw, cb], axis=0)

    row = hyperedge_index[0].astype(jnp.int32)
    col = hyperedge_index[1].astype(jnp.int32)

    return _sc_sheaf(tabx, tabe, row, col, consts, n_inc)
